# baseline (speedup 1.0000x reference)
import jax
import jax.numpy as jnp
from jax import lax
from jax.experimental import pallas as pl
from jax.experimental.pallas import tpu as pltpu
from jax.experimental.pallas import tpu_sc as plsc

L_MAX = 5
FEAT = 16
HEADS = 8
N = 512
E = 8192

E_PAD = 8200
TROWS = E_PAD * L_MAX
B = N * N
NC, NS, LANES = 2, 16, 16
NW = NC * NS
PAIRS_PER_W = B // NW
C = 1024
NCH = PAIRS_PER_W // C
IDX_ROWS = 5 * C // 128
RPL = C // 128


def _proj_body(edata_t_ref, w_ref, out_ref):
    y = lax.dot_general(
        edata_t_ref[:, :], w_ref[:, :],
        (((0,), (1,)), ((), ())),
        preferred_element_type=jnp.float32,
    )
    out_ref[pl.ds(0, E), :] = y
    out_ref[pl.ds(E, E_PAD - E), :] = jnp.zeros(
        (E_PAD - E, L_MAX * HEADS), jnp.float32
    )


def _make_table(edge_feat, embedding_table):
    proj2d = pl.pallas_call(
        _proj_body,
        out_shape=jax.ShapeDtypeStruct((E_PAD, L_MAX * HEADS), jnp.float32),
    )(jnp.transpose(edge_feat), embedding_table)
    return proj2d.reshape(TROWS, HEADS)


def _sc_body(sp_hbm, dist_hbm, tab_hbm, z_hbm, out_hbm,
             tab_sh, spbuf, distbuf, recipbuf, idxbuf, acc, outbuf,
             sem_in0, sem_in1, sem_g0, sem_g1, sem_out0, sem_out1,
             sem_z0, sem_z1):
    sid = lax.axis_index("s")
    wid = lax.axis_index("c") * NS + sid
    base = wid * PAIRS_PER_W
    iota = lax.iota(jnp.int32, LANES)
    expand = lax.shift_right_logical(iota, 3)
    patt_h = jnp.bitwise_and(iota, 7)
    patt_out = patt_h * 128 + expand
    zero16 = iota * 0
    sem_in = [sem_in0, sem_in1]
    sem_g = [sem_g0, sem_g1]
    sem_out = [sem_out0, sem_out1]
    sem_z = [sem_z0, sem_z1]

    def in_copies(g):
        par = g % 2
        pbase = base + g * C
        cps = [
            (sp_hbm.at[pl.ds(l * B + pbase, C)],
             spbuf.at[par].at[pl.ds(l * C, C)], sem_in[par])
            for l in range(L_MAX)
        ]
        cps.append((dist_hbm.at[pl.ds(pbase, C)], distbuf.at[par], sem_in[par]))
        return cps

    def gather_copies(g):
        par = g % 2
        return [
            (tab_sh.at[idxbuf.at[par].at[r]], acc.at[par].at[r % RPL], sem_g[par])
            for r in range(IDX_ROWS)
        ]

    def out_copy(g):
        par = g % 2
        pbase = base + g * C
        return (outbuf.at[par], out_hbm.at[pl.ds(pbase * HEADS, C * HEADS)],
                sem_out[par])

    def build_and_fire(g):
        par = g % 2

        @plsc.parallel_loop(0, C // LANES, 1, unroll=4)
        def recip_body(i):
            v = distbuf[par, pl.ds(i * LANES, LANES)].astype(jnp.float32)
            v = jnp.minimum(jnp.maximum(v, 1.0), float(L_MAX))
            recipbuf[par, pl.ds(i * LANES, LANES)] = 1.0 / v

        for l in range(L_MAX):
            @plsc.parallel_loop(0, C // LANES, 1, unroll=4)
            def idx_body(j, l=l):
                v = spbuf[par, pl.ds(l * C + j * LANES, LANES)]
                idxbuf[par, l * RPL + j // 8, pl.ds((j % 8) * LANES, LANES)] = (
                    v * 5 + l
                )

        pltpu.make_async_copy(z_hbm, acc.at[par], sem_z[par]).wait()
        for cp in gather_copies(g):
            pltpu.async_copy(*cp, add=True)

    def reduce_and_out(g):
        par = g % 2
        for cp in gather_copies(g):
            pltpu.make_async_copy(*cp).wait()
        if g >= 2:
            pltpu.make_async_copy(*out_copy(g - 2)).wait()

        @plsc.parallel_loop(0, C * HEADS // LANES, 1, unroll=8)
        def red_body(p):
            r0 = p // 64
            cidx = expand + (p % 64) * 2
            a = plsc.load_gather(acc.at[par], [zero16 + r0, cidx, patt_h])
            rcp = plsc.load_gather(recipbuf.at[par], [expand + p * 2])
            idxo = patt_out + (
                (p // 256) * (HEADS * N)
                + ((p % 256) // 64) * (HEADS * 128)
                + (p % 64) * 2
            )
            plsc.store_scatter(outbuf.at[par], [idxo], a * rcp)
        pltpu.async_copy(*out_copy(g))
        if g + 2 < NCH:
            pltpu.async_copy(z_hbm, acc.at[par], sem_z[par])

    @pl.when(sid == 0)
    def _():
        pltpu.sync_copy(tab_hbm, tab_sh)

    for cp in in_copies(0):
        pltpu.async_copy(*cp)
    pltpu.async_copy(z_hbm, acc.at[0], sem_z[0])
    pltpu.async_copy(z_hbm, acc.at[1], sem_z[1])
    plsc.subcore_barrier()
    for g in range(NCH):
        for cp in in_copies(g):
            pltpu.make_async_copy(*cp).wait()
        if g + 1 < NCH:
            for cp in in_copies(g + 1):
                pltpu.async_copy(*cp)
        build_and_fire(g)
        if g >= 1:
            reduce_and_out(g - 1)
    reduce_and_out(NCH - 1)
    pltpu.make_async_copy(*out_copy(NCH - 2)).wait()
    pltpu.make_async_copy(*out_copy(NCH - 1)).wait()


@jax.jit
def kernel(edge_feat, shortest_path, shortest_distance, embedding_table):
    table = _make_table(edge_feat, embedding_table)
    sp_lmaj = jnp.transpose(shortest_path, (2, 0, 1)).reshape(-1)
    dist_flat = shortest_distance.reshape(-1)

    mesh = plsc.VectorSubcoreMesh(
        core_axis_name="c", subcore_axis_name="s", num_cores=NC, num_subcores=NS
    )
    run = pl.kernel(
        _sc_body,
        out_type=jax.ShapeDtypeStruct((B * HEADS,), jnp.float32),
        mesh=mesh,
        compiler_params=pltpu.CompilerParams(
            needs_layout_passes=False, use_tc_tiling_on_sc=False
        ),
        scratch_types=[
            pltpu.VMEM_SHARED((TROWS, HEADS), jnp.float32),
            pltpu.VMEM((2, L_MAX * C), jnp.int32),
            pltpu.VMEM((2, C), jnp.int32),
            pltpu.VMEM((2, C), jnp.float32),
            pltpu.VMEM((2, IDX_ROWS, 128), jnp.int32),
            pltpu.VMEM((2, RPL, 128, HEADS), jnp.float32),
            pltpu.VMEM((2, C * HEADS), jnp.float32),
            pltpu.SemaphoreType.DMA,
            pltpu.SemaphoreType.DMA,
            pltpu.SemaphoreType.DMA,
            pltpu.SemaphoreType.DMA,
            pltpu.SemaphoreType.DMA,
            pltpu.SemaphoreType.DMA,
            pltpu.SemaphoreType.DMA,
            pltpu.SemaphoreType.DMA,
        ],
    )
    zeros = jnp.zeros((RPL, 128, HEADS), jnp.float32)
    enc = run(sp_lmaj, dist_flat, table, zeros)
    return (
        enc.reshape(N, N // 128, HEADS, 128)
        .transpose(0, 1, 3, 2)
        .reshape(1, N, N, HEADS)
    )


# speedup vs baseline: 1.0059x; 1.0059x over previous
import jax
import jax.numpy as jnp
from jax import lax
from jax.experimental import pallas as pl
from jax.experimental.pallas import tpu as pltpu
from jax.experimental.pallas import tpu_sc as plsc

L_MAX = 5
FEAT = 16
HEADS = 8
N = 512
E = 8192

E_PAD = 8200
TROWS = E_PAD * L_MAX
B = N * N
NC, NS, LANES = 2, 16, 16
NW = NC * NS
PAIRS_PER_W = B // NW
C = 1024
NCH = PAIRS_PER_W // C
IDX_ROWS = 5 * C // 128
RPL = C // 128


def _proj_body(edata_t_ref, w_ref, out_ref):
    y = lax.dot_general(
        edata_t_ref[:, :], w_ref[:, :],
        (((0,), (1,)), ((), ())),
        preferred_element_type=jnp.float32,
    )
    out_ref[pl.ds(0, E), :] = y
    out_ref[pl.ds(E, E_PAD - E), :] = jnp.zeros(
        (E_PAD - E, L_MAX * HEADS), jnp.float32
    )


def _make_table(edge_feat, embedding_table):
    proj2d = pl.pallas_call(
        _proj_body,
        out_shape=jax.ShapeDtypeStruct((E_PAD, L_MAX * HEADS), jnp.float32),
    )(jnp.transpose(edge_feat), embedding_table)
    return proj2d.reshape(TROWS, HEADS)


def _sc_body(sp_hbm, dist_hbm, tab_hbm, z_hbm, out_hbm,
             tab_sh, spbuf, distbuf, recipbuf, idxbuf, acc, outbuf,
             sem_in0, sem_in1, sem_g0, sem_g1, sem_out0, sem_out1,
             sem_z0, sem_z1):
    sid = lax.axis_index("s")
    wid = lax.axis_index("c") * NS + sid
    base = wid * PAIRS_PER_W
    iota = lax.iota(jnp.int32, LANES)
    expand = lax.shift_right_logical(iota, 3)
    patt_h = jnp.bitwise_and(iota, 7)
    patt_out = patt_h * 128 + expand
    zero16 = iota * 0
    sem_in = [sem_in0, sem_in1]
    sem_g = [sem_g0, sem_g1]
    sem_out = [sem_out0, sem_out1]
    sem_z = [sem_z0, sem_z1]

    def in_copies(g):
        par = g % 2
        pbase = base + g * C
        cps = [
            (sp_hbm.at[pl.ds(l * B + pbase, C)],
             spbuf.at[par].at[pl.ds(l * C, C)], sem_in[par])
            for l in range(L_MAX)
        ]
        cps.append((dist_hbm.at[pl.ds(pbase, C)], distbuf.at[par], sem_in[par]))
        return cps

    def gather_copies(g):
        par = g % 2
        return [
            (tab_sh.at[idxbuf.at[par].at[r]], acc.at[par].at[r % RPL], sem_g[par])
            for r in range(IDX_ROWS)
        ]

    def out_copy(g):
        par = g % 2
        pbase = base + g * C
        return (outbuf.at[par], out_hbm.at[pl.ds(pbase * HEADS, C * HEADS)],
                sem_out[par])

    def build_and_fire(g):
        par = g % 2

        @plsc.parallel_loop(0, C // LANES, 1, unroll=4)
        def recip_body(i):
            v = distbuf[par, pl.ds(i * LANES, LANES)].astype(jnp.float32)
            v = jnp.minimum(jnp.maximum(v, 1.0), float(L_MAX))
            recipbuf[par, pl.ds(i * LANES, LANES)] = 1.0 / v

        for l in range(L_MAX):
            @plsc.parallel_loop(0, C // LANES, 1, unroll=4)
            def idx_body(j, l=l):
                v = spbuf[par, pl.ds(l * C + j * LANES, LANES)]
                idxbuf[par, l * RPL + j // 8, pl.ds((j % 8) * LANES, LANES)] = (
                    v * 5 + l
                )

        pltpu.make_async_copy(z_hbm, acc.at[par], sem_z[par]).wait()
        for cp in gather_copies(g):
            pltpu.async_copy(*cp, add=True)

    def reduce_and_out(g):
        par = g % 2
        for cp in gather_copies(g):
            pltpu.make_async_copy(*cp).wait()
        if g >= 2:
            pltpu.make_async_copy(*out_copy(g - 2)).wait()

        @plsc.parallel_loop(0, C * HEADS // LANES, 1, unroll=4)
        def red_body(p):
            r0 = p // 64
            cidx = expand + (p % 64) * 2
            a = plsc.load_gather(acc.at[par], [zero16 + r0, cidx, patt_h])
            rcp = plsc.load_gather(recipbuf.at[par], [expand + p * 2])
            idxo = patt_out + (
                (p // 256) * (HEADS * N)
                + ((p % 256) // 64) * (HEADS * 128)
                + (p % 64) * 2
            )
            plsc.store_scatter(outbuf.at[par], [idxo], a * rcp)
        pltpu.async_copy(*out_copy(g))
        if g + 2 < NCH:
            pltpu.async_copy(z_hbm, acc.at[par], sem_z[par])

    @pl.when(sid == 0)
    def _():
        pltpu.sync_copy(tab_hbm, tab_sh)

    for cp in in_copies(0):
        pltpu.async_copy(*cp)
    pltpu.async_copy(z_hbm, acc.at[0], sem_z[0])
    pltpu.async_copy(z_hbm, acc.at[1], sem_z[1])
    plsc.subcore_barrier()
    for g in range(NCH):
        for cp in in_copies(g):
            pltpu.make_async_copy(*cp).wait()
        if g + 1 < NCH:
            for cp in in_copies(g + 1):
                pltpu.async_copy(*cp)
        build_and_fire(g)
        if g >= 1:
            reduce_and_out(g - 1)
    reduce_and_out(NCH - 1)
    pltpu.make_async_copy(*out_copy(NCH - 2)).wait()
    pltpu.make_async_copy(*out_copy(NCH - 1)).wait()


@jax.jit
def kernel(edge_feat, shortest_path, shortest_distance, embedding_table):
    table = _make_table(edge_feat, embedding_table)
    sp_lmaj = jnp.transpose(shortest_path, (2, 0, 1)).reshape(-1)
    dist_flat = shortest_distance.reshape(-1)

    mesh = plsc.VectorSubcoreMesh(
        core_axis_name="c", subcore_axis_name="s", num_cores=NC, num_subcores=NS
    )
    run = pl.kernel(
        _sc_body,
        out_type=jax.ShapeDtypeStruct((B * HEADS,), jnp.float32),
        mesh=mesh,
        compiler_params=pltpu.CompilerParams(
            needs_layout_passes=False, use_tc_tiling_on_sc=False
        ),
        scratch_types=[
            pltpu.VMEM_SHARED((TROWS, HEADS), jnp.float32),
            pltpu.VMEM((2, L_MAX * C), jnp.int32),
            pltpu.VMEM((2, C), jnp.int32),
            pltpu.VMEM((2, C), jnp.float32),
            pltpu.VMEM((2, IDX_ROWS, 128), jnp.int32),
            pltpu.VMEM((2, RPL, 128, HEADS), jnp.float32),
            pltpu.VMEM((2, C * HEADS), jnp.float32),
            pltpu.SemaphoreType.DMA,
            pltpu.SemaphoreType.DMA,
            pltpu.SemaphoreType.DMA,
            pltpu.SemaphoreType.DMA,
            pltpu.SemaphoreType.DMA,
            pltpu.SemaphoreType.DMA,
            pltpu.SemaphoreType.DMA,
            pltpu.SemaphoreType.DMA,
        ],
    )
    zeros = jnp.zeros((RPL, 128, HEADS), jnp.float32)
    enc = run(sp_lmaj, dist_flat, table, zeros)
    return (
        enc.reshape(N, N // 128, HEADS, 128)
        .transpose(0, 1, 3, 2)
        .reshape(1, N, N, HEADS)
    )
